# baseline (speedup 1.0000x reference)
import functools

import jax
import jax.numpy as jnp
from jax import lax
from jax.experimental import pallas as pl
from jax.experimental.pallas import tpu as pltpu
from jax.experimental.pallas import tpu_sc as plsc

V = 1000000
D = 64
DW = 128
B = 16384 * 50
NC, NS = 2, 16
NW = NC * NS
BPW = B // NW
CH = 128
NCHUNK = BPW // CH
NB = 5
NMAIN = NCHUNK - NB


@functools.lru_cache(maxsize=1)
def _build():
    mesh = plsc.VectorSubcoreMesh(core_axis_name="c", subcore_axis_name="s")

    @functools.partial(
        pl.kernel,
        mesh=mesh,
        out_type=jax.ShapeDtypeStruct((B, DW), jnp.float32),
        scratch_types=(
            [pltpu.VMEM((BPW,), jnp.int32)]
            + [pltpu.VMEM((CH, DW), jnp.float32) for _ in range(NB)]
            + [pltpu.SemaphoreType.DMA for _ in range(2 * NB)]
        ),
    )
    def gather(wide_hbm, idx_hbm, out_hbm, idx_v, *bufs_and_sems):
        rows = bufs_and_sems[:NB]
        gsem = bufs_and_sems[NB:2 * NB]
        osem = bufs_and_sems[2 * NB:]

        wid = lax.axis_index("s") * NC + lax.axis_index("c")
        wbase = wid * BPW
        pltpu.sync_copy(idx_hbm.at[pl.ds(wbase, BPW)], idx_v)

        def start_gather(g, b):
            pltpu.async_copy(
                wide_hbm.at[idx_v.at[pl.ds(g * CH, CH)]], rows[b], gsem[b])

        def wait_gather(b):
            pltpu.make_async_copy(
                wide_hbm.at[pl.ds(0, CH)], rows[b], gsem[b]).wait()

        def start_out(g, b):
            pltpu.async_copy(
                rows[b], out_hbm.at[pl.ds(wbase + g * CH, CH)], osem[b])

        def wait_out(g, b):
            pltpu.make_async_copy(
                rows[b], out_hbm.at[pl.ds(wbase + g * CH, CH)], osem[b]).wait()

        for b in range(NB):
            start_gather(b, b)

        def main(i, carry):
            g0 = i * NB
            for b in range(NB):
                g = g0 + b
                wait_gather(b)
                start_out(g, b)
                wait_out(g, b)
                start_gather(g + NB, b)
            return carry

        lax.fori_loop(0, NMAIN // NB, main, 0)

        for b in range(NB):
            g = NMAIN + b
            wait_gather(b)
            pltpu.sync_copy(rows[b], out_hbm.at[pl.ds(wbase + g * CH, CH)])

    return gather


PR = 8000


def _pad_block(m_ref, o_ref):
    o_ref[...] = jnp.concatenate(
        [m_ref[...], jnp.zeros((PR, DW - D), jnp.float32)], axis=1)


def _widen(mat):
    return pl.pallas_call(
        _pad_block,
        grid=(V // PR,),
        in_specs=[pl.BlockSpec((PR, D), lambda i: (i, 0))],
        out_specs=pl.BlockSpec((PR, DW), lambda i: (i, 0)),
        out_shape=jax.ShapeDtypeStruct((V, DW), jnp.float32),
    )(mat)


def kernel(x, mat):
    idx = x.reshape(-1).astype(jnp.int32)
    wide = _widen(mat)
    out_w = _build()(wide, idx)
    return out_w[:, :D]


# speedup vs baseline: 1.1364x; 1.1364x over previous
import functools

import jax
import jax.numpy as jnp
from jax import lax
from jax.experimental import pallas as pl
from jax.experimental.pallas import tpu as pltpu
from jax.experimental.pallas import tpu_sc as plsc

V = 1000000
D = 64
DW = 128
B = 16384 * 50
NC, NS = 2, 16
NW = NC * NS
BPW = B // NW
CH = 128
NCHUNK = BPW // CH
NB = 5
NMAIN = NCHUNK - NB


@functools.lru_cache(maxsize=1)
def _build():
    mesh = plsc.VectorSubcoreMesh(core_axis_name="c", subcore_axis_name="s")

    @functools.partial(
        pl.kernel,
        mesh=mesh,
        out_type=jax.ShapeDtypeStruct((B, DW), jnp.float32),
        scratch_types=(
            [pltpu.VMEM((BPW,), jnp.int32)]
            + [pltpu.VMEM((CH, DW), jnp.float32) for _ in range(NB)]
            + [pltpu.SemaphoreType.DMA for _ in range(2 * NB)]
        ),
    )
    def gather(wide_hbm, idx_hbm, out_hbm, idx_v, *bufs_and_sems):
        rows = bufs_and_sems[:NB]
        gsem = bufs_and_sems[NB:2 * NB]
        osem = bufs_and_sems[2 * NB:]

        wid = lax.axis_index("s") * NC + lax.axis_index("c")
        wbase = wid * BPW
        pltpu.sync_copy(idx_hbm.at[pl.ds(wbase, BPW)], idx_v)

        def start_gather(g, b):
            pltpu.async_copy(
                wide_hbm.at[idx_v.at[pl.ds(g * CH, CH)]], rows[b], gsem[b])

        def wait_gather(b):
            pltpu.make_async_copy(
                wide_hbm.at[pl.ds(0, CH)], rows[b], gsem[b]).wait()

        def start_out(g, b):
            pltpu.async_copy(
                rows[b], out_hbm.at[pl.ds(wbase + g * CH, CH)], osem[b])

        def wait_out(g, b):
            pltpu.make_async_copy(
                rows[b], out_hbm.at[pl.ds(wbase + g * CH, CH)], osem[b]).wait()

        for b in range(NB):
            start_gather(b, b)

        def main(i, carry):
            g0 = i * NB
            for b in range(NB):
                g = g0 + b
                wait_gather(b)
                start_out(g, b)
                wait_out(g, b)
                start_gather(g + NB, b)
            return carry

        lax.fori_loop(0, NMAIN // NB, main, 0)

        for b in range(NB):
            g = NMAIN + b
            wait_gather(b)
            pltpu.sync_copy(rows[b], out_hbm.at[pl.ds(wbase + g * CH, CH)])

    return gather


def kernel(x, mat):
    idx = x.reshape(-1).astype(jnp.int32)
    wide = jnp.pad(mat, ((0, 0), (0, DW - D)))
    out_w = _build()(wide, idx)
    return out_w[:, :D]
